# Initial kernel scaffold; baseline (speedup 1.0000x reference)
import jax
import jax.numpy as jnp
from jax.experimental import pallas as pl


def kernel(faces, vertices, face_vertices, angles, face_areas, normals, face_mask, vertex_table, angle_table, area_table, normal_table, pe, W_face, b_face, W_enc, b_enc, codebook):
    raise NotImplementedError("write your pallas kernel here")


# speedup vs baseline: 1.5393x; 1.5393x over previous
import functools
import math

import jax
import jax.numpy as jnp
from jax import lax
from jax.experimental import pallas as pl
from jax.experimental.pallas import tpu as pltpu
from jax.experimental.pallas import tpu_sc as plsc

B, F, NT = 4, 2048, 8192
ND = 128
DV = 192
NC = 16384
TOK_TILE = 512
CB_CHUNK = 2048
N_TILES = NT // TOK_TILE
N_CHUNKS = NC // CB_CHUNK


def _enc_body(fv_ref, ang_ref, ar_ref, nor_ref, mask_ref, pe_ref,
              vtab_ref, atab_ref, artab_ref, ntab_ref,
              wf_ref, bf_ref, we_ref, be_ref, out_ref):
    def disc(x, mn, mx):
        y = (x - mn) / (mx - mn)
        y = y * (ND - 1)
        return jnp.round(y).astype(jnp.int32)

    iota = lax.broadcasted_iota(jnp.int32, (TOK_TILE, ND), 1)

    def onehot_mm(col_idx, tab):
        oh = (col_idx == iota).astype(jnp.float32)
        return lax.dot_general(oh, tab, (((1,), (0,)), ((), ())),
                               precision=lax.Precision.HIGHEST,
                               preferred_element_type=jnp.float32)

    pe = pe_ref[...]
    vtab = vtab_ref[...]
    ntab = ntab_ref[...]
    atab = atab_ref[...]
    artab = artab_ref[...]

    vd = disc(fv_ref[...], -1.0, 1.0)
    nrd = disc(nor_ref[...], -1.0, 1.0)
    ad = disc(ang_ref[...], 0.0, math.pi)
    ard = disc(ar_ref[...], 0.0, 4.0)

    pieces = []
    for s in range(9):
        pieces.append(onehot_mm(vd[:, s:s + 1], vtab) + pe)
    for s in range(3):
        pieces.append(onehot_mm(nrd[:, s:s + 1], ntab))
    for s in range(3):
        pieces.append(onehot_mm(ad[:, s:s + 1], atab))
    pieces.append(onehot_mm(ard[:, 0:1], artab))
    graph = jnp.concatenate(pieces, axis=1)

    graph = graph.astype(jnp.bfloat16).astype(jnp.float32)
    g = lax.dot_general(graph, wf_ref[...], (((1,), (1,)), ((), ())),
                        preferred_element_type=jnp.float32) + bf_ref[...][None, :]
    g = jnp.where(mask_ref[...] != 0.0, g, 0.0)
    g = g.astype(jnp.bfloat16).astype(jnp.float32)
    enc = lax.dot_general(g, we_ref[...], (((1,), (1,)), ((), ())),
                          preferred_element_type=jnp.float32) + be_ref[...][None, :]
    out_ref[...] = enc


def _enc_pallas(fv, ang, ar, nor, maskf, pe2048, vtab, atab, artab, ntab,
                wf, bf, we, be):
    full = lambda shape: pl.BlockSpec(shape, lambda i: tuple(0 for _ in shape))
    return pl.pallas_call(
        _enc_body,
        grid=(N_TILES,),
        in_specs=[
            pl.BlockSpec((TOK_TILE, 9), lambda i: (i, 0)),
            pl.BlockSpec((TOK_TILE, 3), lambda i: (i, 0)),
            pl.BlockSpec((TOK_TILE, 1), lambda i: (i, 0)),
            pl.BlockSpec((TOK_TILE, 3), lambda i: (i, 0)),
            pl.BlockSpec((TOK_TILE, 1), lambda i: (i, 0)),
            pl.BlockSpec((TOK_TILE, 64), lambda i: (i % (F // TOK_TILE), 0)),
            full((ND, 64)),
            full((ND, 16)),
            full((ND, 16)),
            full((ND, 64)),
            full((196, 832)),
            full((196,)),
            full((DV, 196)),
            full((DV,)),
        ],
        out_specs=pl.BlockSpec((TOK_TILE, DV), lambda i: (i, 0)),
        out_shape=jax.ShapeDtypeStruct((NT, DV), jnp.float32),
    )(fv, ang, ar, nor, maskf, pe2048, vtab, atab, artab, ntab, wf, bf, we, be)


XLA_CHUNKS = ((0, 5504), (5504, 5504), (11008, 5376))


def _make_score_body(bf16_carry):
    def _score_body(r_ref, rsq_ref, cb_ref, cbsq_ref, idx_ref):
        r = r_ref[...]
        rsq = rsq_ref[...][:, None]
        acc_v = None
        acc_i = None
        for start, size in XLA_CHUNKS:
            cbc = cb_ref[pl.ds(start, size), :]
            s = lax.dot_general(r, cbc, (((1,), (1,)), ((), ())),
                                preferred_element_type=jnp.float32)
            d = (rsq - 2.0 * s) + cbsq_ref[pl.ds(start, size)][None, :]
            m = jnp.min(d, axis=1)
            iota = lax.broadcasted_iota(jnp.int32, (TOK_TILE, size), 1)
            i_in = jnp.min(jnp.where(d == m[:, None], iota, NC), axis=1)
            gi = i_in + start
            if acc_v is None:
                acc_v, acc_i = m, gi
            else:
                take = m < acc_v
                acc_v = jnp.where(take, m, acc_v)
                acc_i = jnp.where(take, gi, acc_i)
            if bf16_carry:
                acc_v = acc_v.astype(jnp.bfloat16).astype(jnp.float32)
        idx_ref[...] = acc_i
    return _score_body


def _score_pallas(r, rsq, codebook, cb_sq, bf16_carry):
    return pl.pallas_call(
        _make_score_body(bf16_carry),
        grid=(N_TILES,),
        in_specs=[
            pl.BlockSpec((TOK_TILE, DV), lambda i: (i, 0)),
            pl.BlockSpec((TOK_TILE,), lambda i: (i,)),
            pl.BlockSpec((NC, DV), lambda i: (0, 0)),
            pl.BlockSpec((NC,), lambda i: (0,)),
        ],
        out_specs=pl.BlockSpec((TOK_TILE,), lambda i: (i,)),
        out_shape=jax.ShapeDtypeStruct((NT,), jnp.int32),
    )(r, rsq, codebook, cb_sq)


DVP = 256


def _sc_gather(idx, codebook_pad):
    info = plsc.get_sparse_core_info()
    nw = info.num_cores * info.num_subcores
    per_w = NT // nw
    k = 128
    nk = per_w // k
    mesh = plsc.VectorSubcoreMesh(core_axis_name="c", subcore_axis_name="s")

    @functools.partial(
        pl.kernel, mesh=mesh,
        out_type=jax.ShapeDtypeStruct((NT, DVP), jnp.float32),
        scratch_types=[
            pltpu.VMEM((nk, k), jnp.int32),
            pltpu.VMEM((k, DVP), jnp.float32),
            pltpu.SemaphoreType.DMA,
        ],
    )
    def gather_k(idx_hbm, table_hbm, out_hbm, idx_v, rows_v, sem):
        wid = lax.axis_index("s") * info.num_cores + lax.axis_index("c")
        base = wid * per_w
        pltpu.sync_copy(idx_hbm.at[wid], idx_v)
        for j in range(nk):
            pltpu.async_copy(table_hbm.at[idx_v.at[j]], rows_v, sem).wait()
            pltpu.sync_copy(rows_v, out_hbm.at[pl.ds(base + j * k, k)])

    return gather_k(idx.reshape(nw, nk, k), codebook_pad)


def _combine_body(enc_ref, q1_ref, q2_ref, out_ref, loss_ref):
    enc = enc_ref[...]
    q = q1_ref[...] + q2_ref[...]
    out_ref[...] = enc + (q - enc)
    e = enc - q
    part = jnp.sum(e * e)
    prev = jnp.where(pl.program_id(0) == 0, 0.0, loss_ref[...])
    tot = prev + part
    last = pl.program_id(0) == pl.num_programs(0) - 1
    loss_ref[...] = jnp.where(last, tot / (NT * DV), tot)


def _combine_pallas(enc, q1, q2):
    return pl.pallas_call(
        _combine_body,
        grid=(N_TILES,),
        in_specs=[pl.BlockSpec((TOK_TILE, DV), lambda i: (i, 0))] * 3,
        out_specs=[
            pl.BlockSpec((TOK_TILE, DV), lambda i: (i, 0)),
            pl.BlockSpec((1, 1), lambda i: (0, 0)),
        ],
        out_shape=[
            jax.ShapeDtypeStruct((NT, DV), jnp.float32),
            jax.ShapeDtypeStruct((1, 1), jnp.float32),
        ],
    )(enc, q1, q2)


def kernel(faces, vertices, face_vertices, angles, face_areas, normals,
           face_mask, vertex_table, angle_table, area_table, normal_table,
           pe, W_face, b_face, W_enc, b_enc, codebook):
    fv = face_vertices.reshape(NT, 9)
    ang = angles.reshape(NT, 3)
    ar = face_areas.reshape(NT, 1)
    nor = normals.reshape(NT, 3)
    maskf = face_mask.reshape(NT, 1).astype(jnp.float32)
    pe2048 = pe[:F]

    enc = _enc_pallas(fv, ang, ar, nor, maskf, pe2048,
                      vertex_table, angle_table, area_table, normal_table,
                      W_face, b_face, W_enc, b_enc)

    cb_sq = jnp.sum(codebook ** 2, axis=-1)
    cb_pad = jnp.pad(codebook, ((0, 0), (0, DVP - DV)))
    rsq1 = jnp.sum(enc.reshape(B, F, DV) ** 2, axis=-1, keepdims=True).reshape(NT)
    idx1 = _score_pallas(enc, rsq1, codebook, cb_sq, True)
    q1 = _sc_gather(idx1, cb_pad)[:, :DV]

    res2 = enc - q1
    rsq2 = jnp.sum(res2.reshape(B, F, DV) ** 2, axis=-1, keepdims=True).reshape(NT)
    idx2 = _score_pallas(res2, rsq2, codebook, cb_sq, False)
    q2 = _sc_gather(idx2, cb_pad)[:, :DV]

    qste, loss = _combine_pallas(enc, q1, q2)

    codes = jnp.stack([idx1.reshape(B, F), idx2.reshape(B, F)], axis=-1)
    return qste.reshape(B, F, DV), codes, loss[0, 0]
